# baseline (speedup 1.0000x reference)
import jax
import jax.numpy as jnp
from jax import lax
from jax.experimental import pallas as pl
from jax.experimental.pallas import tpu as pltpu
from jax.experimental.pallas import tpu_sc as plsc

_B, _S, _F = 1024, 512, 64
_NC, _NS = 2, 16
_NW = _NC * _NS
_ROWS_PER_W = _B // _NW
_CHUNK = 128
_NCHUNK = _S // _CHUNK
_LANES = 16
_NLINES = _S // 2
_LPC = _CHUNK // 2


def _embed_body(tokens_hbm, table_hbm, pos_hbm, out_hbm,
                idx_v, rows_v, store_v, pos_v, gsem, ssem):
    wid = lax.axis_index("s") * _NC + lax.axis_index("c")
    base = wid * _ROWS_PER_W

    pltpu.sync_copy(pos_hbm, pos_v)

    def start_gather(j, b):
        pltpu.sync_copy(tokens_hbm.at[base + j], idx_v.at[b])
        for t in range(_NCHUNK):
            pltpu.async_copy(table_hbm.at[idx_v.at[b, t]], rows_v.at[b, t],
                             gsem.at[b])

    def wait_gather(b):
        for t in range(_NCHUNK):
            pltpu.make_async_copy(table_hbm.at[idx_v.at[b, t]],
                                  rows_v.at[b, t], gsem.at[b]).wait()

    def wait_store(sb):
        pltpu.make_async_copy(store_v.at[sb], out_hbm.at[0, 0],
                              ssem.at[sb]).wait()

    start_gather(0, 0)

    @pl.loop(0, _ROWS_PER_W)
    def _row(j):
        b = j & 1

        @pl.when(j < _ROWS_PER_W - 1)
        def _():
            start_gather(j + 1, 1 - b)

        wait_gather(b)

        for t in range(_NCHUNK):
            sb = t & 1

            if t < 2:
                @pl.when(j > 0)
                def _():
                    wait_store(sb)
            else:
                wait_store(sb)

            @pl.loop(0, _LPC, unroll=2)
            def _add_pos(l):
                for h in range(2):
                    for k in range(_F // _LANES):
                        dsl = pl.ds(h * _F + k * _LANES, _LANES)
                        x = rows_v[b, t, 2 * l + h, pl.ds(k * _LANES, _LANES)]
                        store_v[sb, l, dsl] = x + pos_v[t, l, dsl]

            pltpu.async_copy(store_v.at[sb], out_hbm.at[base + j, t],
                             ssem.at[sb])

    for sb in range(2):
        wait_store(sb)


def _transpose_body(in_ref, out_ref):
    x = in_ref[0].reshape(_NLINES, 128)
    out_ref[0, :, : _S // 2] = x[:, : _F].T
    out_ref[0, :, _S // 2:] = x[:, _F:].T


def kernel(tokens, node_span_starts, node_span_ends, embed_table, pos_emb):
    del node_span_starts, node_span_ends
    tokens_il = jnp.stack(
        [tokens[:, :_NLINES], tokens[:, _NLINES:]], axis=-1).reshape(_B, _S)
    tokens3 = tokens_il.reshape(_B, _NCHUNK, _CHUNK)
    pos2 = pos_emb.reshape(_S, _F)
    pos_slab = jnp.concatenate(
        [pos2[:_NLINES], pos2[_NLINES:]], axis=1).reshape(_NCHUNK, _LPC, 128)

    mesh = plsc.VectorSubcoreMesh(core_axis_name="c", subcore_axis_name="s")
    lin4 = pl.kernel(
        _embed_body,
        out_type=jax.ShapeDtypeStruct((_B, _NCHUNK, _LPC, 128), jnp.float32),
        mesh=mesh,
        compiler_params=pltpu.CompilerParams(use_tc_tiling_on_sc=False),
        scratch_types=[
            pltpu.VMEM((2, _NCHUNK, _CHUNK), jnp.int32),
            pltpu.VMEM((2, _NCHUNK, _CHUNK, _F), jnp.float32),
            pltpu.VMEM((2, _LPC, 128), jnp.float32),
            pltpu.VMEM((_NCHUNK, _LPC, 128), jnp.float32),
            pltpu.SemaphoreType.DMA((2,)),
            pltpu.SemaphoreType.DMA((2,)),
        ],
    )(tokens3, embed_table, pos_slab)

    out_t = pl.pallas_call(
        _transpose_body,
        grid=(_B,),
        in_specs=[pl.BlockSpec((1, _NCHUNK, _LPC, 128),
                               lambda b: (b, 0, 0, 0))],
        out_specs=pl.BlockSpec((1, _F, _S), lambda b: (b, 0, 0)),
        out_shape=jax.ShapeDtypeStruct((_B, _F, _S), jnp.float32),
    )(lin4)
    return out_t.swapaxes(1, 2)


# speedup vs baseline: 2.6623x; 2.6623x over previous
import jax
import jax.numpy as jnp
from jax import lax
from jax.experimental import pallas as pl
from jax.experimental.pallas import tpu as pltpu
from jax.experimental.pallas import tpu_sc as plsc

_B, _S, _F = 1024, 512, 64
_NC, _NS = 2, 16
_NW = _NC * _NS
_ROWS_PER_W = _B // _NW
_CHUNK = 128
_NCHUNK = _S // _CHUNK
_LANES = 16
_NLINES = _S // 2
_LPC = _CHUNK // 2


def _embed_body(tokens_hbm, table_hbm, pos_hbm, out_hbm,
                idx_v, rows_v, store_v, pos_v, gsem, ssem):
    wid = lax.axis_index("s") * _NC + lax.axis_index("c")
    base = wid * _ROWS_PER_W

    pltpu.sync_copy(pos_hbm, pos_v)

    def start_gather(j, b):
        pltpu.sync_copy(tokens_hbm.at[base + j], idx_v.at[b])
        for t in range(_NCHUNK):
            pltpu.async_copy(table_hbm.at[idx_v.at[b, t]], rows_v.at[b, t],
                             gsem.at[b])

    def wait_gather(b):
        for t in range(_NCHUNK):
            pltpu.make_async_copy(table_hbm.at[idx_v.at[b, t]],
                                  rows_v.at[b, t], gsem.at[b]).wait()

    def wait_store(sb):
        pltpu.make_async_copy(store_v.at[sb], out_hbm.at[0, 0],
                              ssem.at[sb]).wait()

    def process_row(j, b, first):
        wait_gather(b)
        for t in range(_NCHUNK):
            sb = t & 1

            if t < 2 and first:
                @pl.when(j > 0)
                def _():
                    wait_store(sb)
            else:
                wait_store(sb)

            @pl.loop(0, _LPC, unroll=2)
            def _add_pos(l):
                for h in range(2):
                    for k in range(_F // _LANES):
                        dsl = pl.ds(h * _F + k * _LANES, _LANES)
                        x = rows_v[b, t, 2 * l + h, pl.ds(k * _LANES, _LANES)]
                        store_v[sb, l, dsl] = x + pos_v[t, l, dsl]

            pltpu.async_copy(store_v.at[sb], out_hbm.at[base + j, t],
                             ssem.at[sb])

    start_gather(0, 0)

    @pl.loop(0, _ROWS_PER_W // 2)
    def _row_pair(i):
        j0 = 2 * i
        start_gather(j0 + 1, 1)
        process_row(j0, 0, first=True)

        @pl.when(j0 + 2 < _ROWS_PER_W)
        def _():
            start_gather(j0 + 2, 0)

        process_row(j0 + 1, 1, first=False)

    for sb in range(2):
        wait_store(sb)


_TCB = 8


def _transpose_body(in_ref, out_ref):
    x = in_ref[...].reshape(_TCB, _NLINES, 128)
    i0 = lax.broadcasted_iota(jnp.int32, (_NLINES, _NLINES), 0)
    i1 = lax.broadcasted_iota(jnp.int32, (_NLINES, _NLINES), 1)
    ident = (i0 == i1).astype(jnp.float32)
    xt = lax.dot_general(x, ident, (((1,), (0,)), ((), ())),
                         preferred_element_type=jnp.float32)
    out_ref[:, :, : _S // 2] = xt[:, : _F, :]
    out_ref[:, :, _S // 2:] = xt[:, _F:, :]


def kernel(tokens, node_span_starts, node_span_ends, embed_table, pos_emb):
    del node_span_starts, node_span_ends
    tokens_il = jnp.stack(
        [tokens[:, :_NLINES], tokens[:, _NLINES:]], axis=-1).reshape(_B, _S)
    tokens3 = tokens_il.reshape(_B, _NCHUNK, _CHUNK)
    pos2 = pos_emb.reshape(_S, _F)
    pos_slab = jnp.concatenate(
        [pos2[:_NLINES], pos2[_NLINES:]], axis=1).reshape(_NCHUNK, _LPC, 128)

    mesh = plsc.VectorSubcoreMesh(core_axis_name="c", subcore_axis_name="s")
    lin4 = pl.kernel(
        _embed_body,
        out_type=jax.ShapeDtypeStruct((_B, _NCHUNK, _LPC, 128), jnp.float32),
        mesh=mesh,
        compiler_params=pltpu.CompilerParams(use_tc_tiling_on_sc=False),
        scratch_types=[
            pltpu.VMEM((2, _NCHUNK, _CHUNK), jnp.int32),
            pltpu.VMEM((2, _NCHUNK, _CHUNK, _F), jnp.float32),
            pltpu.VMEM((2, _LPC, 128), jnp.float32),
            pltpu.VMEM((_NCHUNK, _LPC, 128), jnp.float32),
            pltpu.SemaphoreType.DMA((2,)),
            pltpu.SemaphoreType.DMA((2,)),
        ],
    )(tokens3, embed_table, pos_slab)

    out_t = pl.pallas_call(
        _transpose_body,
        grid=(_B // _TCB,),
        in_specs=[pl.BlockSpec((_TCB, _NCHUNK, _LPC, 128),
                               lambda b: (b, 0, 0, 0))],
        out_specs=pl.BlockSpec((_TCB, _F, _S), lambda b: (b, 0, 0)),
        out_shape=jax.ShapeDtypeStruct((_B, _F, _S), jnp.float32),
    )(lin4)
    return out_t.swapaxes(1, 2)


# speedup vs baseline: 2.9425x; 1.1052x over previous
import jax
import jax.numpy as jnp
from jax import lax
from jax.experimental import pallas as pl
from jax.experimental.pallas import tpu as pltpu
from jax.experimental.pallas import tpu_sc as plsc

_B, _S, _F = 1024, 512, 64
_NC, _NS = 2, 16
_NW = _NC * _NS
_ROWS_PER_W = _B // _NW
_CHUNK = 128
_NCHUNK = _S // _CHUNK
_LANES = 16
_NLINES = _S // 2
_LPC = _CHUNK // 2


def _embed_body(tokens_hbm, table_hbm, pos_hbm, out_hbm,
                idx_v, rows_v, store_v, pos_v, gsem, ssem):
    wid = lax.axis_index("s") * _NC + lax.axis_index("c")
    base = wid * _ROWS_PER_W

    pltpu.sync_copy(pos_hbm, pos_v)

    def start_gather(j, b):
        pltpu.sync_copy(tokens_hbm.at[base + j], idx_v.at[b])
        for t in range(_NCHUNK):
            pltpu.async_copy(table_hbm.at[idx_v.at[b, t]], rows_v.at[b, t],
                             gsem.at[b])

    def wait_gather(b):
        for t in range(_NCHUNK):
            pltpu.make_async_copy(table_hbm.at[idx_v.at[b, t]],
                                  rows_v.at[b, t], gsem.at[b]).wait()

    def wait_store(sb):
        pltpu.make_async_copy(store_v.at[sb], out_hbm.at[0, 0],
                              ssem.at[sb]).wait()

    def process_row(j, b, first):
        wait_gather(b)
        for t in range(_NCHUNK):
            sb = t & 1

            if t < 2 and first:
                @pl.when(j > 0)
                def _():
                    wait_store(sb)
            else:
                wait_store(sb)

            @pl.loop(0, _LPC, unroll=4)
            def _add_pos(l):
                for h in range(2):
                    for k in range(_F // _LANES):
                        dsl = pl.ds(h * _F + k * _LANES, _LANES)
                        x = rows_v[b, t, 2 * l + h, pl.ds(k * _LANES, _LANES)]
                        store_v[sb, l, dsl] = x + pos_v[t, l, dsl]

            pltpu.async_copy(store_v.at[sb], out_hbm.at[base + j, t],
                             ssem.at[sb])

    start_gather(0, 0)

    @pl.loop(0, _ROWS_PER_W // 2)
    def _row_pair(i):
        j0 = 2 * i
        start_gather(j0 + 1, 1)
        process_row(j0, 0, first=True)

        @pl.when(j0 + 2 < _ROWS_PER_W)
        def _():
            start_gather(j0 + 2, 0)

        process_row(j0 + 1, 1, first=False)

    for sb in range(2):
        wait_store(sb)


_TCB = 16


def _transpose_body(in_ref, out_ref):
    x = in_ref[...].reshape(_TCB, _NLINES, 128)
    i0 = lax.broadcasted_iota(jnp.int32, (_NLINES, _NLINES), 0)
    i1 = lax.broadcasted_iota(jnp.int32, (_NLINES, _NLINES), 1)
    ident = (i0 == i1).astype(jnp.float32)
    xt = lax.dot_general(x, ident, (((1,), (0,)), ((), ())),
                         preferred_element_type=jnp.float32)
    out_ref[:, :, : _S // 2] = xt[:, : _F, :]
    out_ref[:, :, _S // 2:] = xt[:, _F:, :]


def kernel(tokens, node_span_starts, node_span_ends, embed_table, pos_emb):
    del node_span_starts, node_span_ends
    tokens_il = jnp.stack(
        [tokens[:, :_NLINES], tokens[:, _NLINES:]], axis=-1).reshape(_B, _S)
    tokens3 = tokens_il.reshape(_B, _NCHUNK, _CHUNK)
    pos2 = pos_emb.reshape(_S, _F)
    pos_slab = jnp.concatenate(
        [pos2[:_NLINES], pos2[_NLINES:]], axis=1).reshape(_NCHUNK, _LPC, 128)

    mesh = plsc.VectorSubcoreMesh(core_axis_name="c", subcore_axis_name="s")
    lin4 = pl.kernel(
        _embed_body,
        out_type=jax.ShapeDtypeStruct((_B, _NCHUNK, _LPC, 128), jnp.float32),
        mesh=mesh,
        compiler_params=pltpu.CompilerParams(use_tc_tiling_on_sc=False),
        scratch_types=[
            pltpu.VMEM((2, _NCHUNK, _CHUNK), jnp.int32),
            pltpu.VMEM((2, _NCHUNK, _CHUNK, _F), jnp.float32),
            pltpu.VMEM((2, _LPC, 128), jnp.float32),
            pltpu.VMEM((_NCHUNK, _LPC, 128), jnp.float32),
            pltpu.SemaphoreType.DMA((2,)),
            pltpu.SemaphoreType.DMA((2,)),
        ],
    )(tokens3, embed_table, pos_slab)

    out_t = pl.pallas_call(
        _transpose_body,
        grid=(_B // _TCB,),
        in_specs=[pl.BlockSpec((_TCB, _NCHUNK, _LPC, 128),
                               lambda b: (b, 0, 0, 0))],
        out_specs=pl.BlockSpec((_TCB, _F, _S), lambda b: (b, 0, 0)),
        out_shape=jax.ShapeDtypeStruct((_B, _F, _S), jnp.float32),
    )(lin4)
    return out_t.swapaxes(1, 2)


# speedup vs baseline: 3.0202x; 1.0264x over previous
import jax
import jax.numpy as jnp
from jax import lax
from jax.experimental import pallas as pl
from jax.experimental.pallas import tpu as pltpu
from jax.experimental.pallas import tpu_sc as plsc

_B, _S, _F = 1024, 512, 64
_NC, _NS = 2, 16
_NW = _NC * _NS
_ROWS_PER_W = _B // _NW
_CHUNK = 128
_NCHUNK = _S // _CHUNK
_LANES = 16
_NLINES = _S // 2
_LPC = _CHUNK // 2


def _embed_body(tokens_hbm, table_hbm, pos_hbm, out_hbm,
                idx_v, rows_v, store_v, pos_v, gsem, ssem):
    wid = lax.axis_index("s") * _NC + lax.axis_index("c")
    base = wid * _ROWS_PER_W

    pltpu.sync_copy(pos_hbm, pos_v)

    def start_gather(j, b):
        pltpu.sync_copy(tokens_hbm.at[base + j], idx_v.at[b])
        for t in range(_NCHUNK):
            pltpu.async_copy(table_hbm.at[idx_v.at[b, t]], rows_v.at[b, t],
                             gsem.at[b])

    def wait_gather(b):
        for t in range(_NCHUNK):
            pltpu.make_async_copy(table_hbm.at[idx_v.at[b, t]],
                                  rows_v.at[b, t], gsem.at[b]).wait()

    def wait_store(sb):
        pltpu.make_async_copy(store_v.at[sb], out_hbm.at[0, 0],
                              ssem.at[sb]).wait()

    def process_row(j, b, first):
        wait_gather(b)
        for t in range(_NCHUNK):
            sb = t & 1

            if t < 2 and first:
                @pl.when(j > 0)
                def _():
                    wait_store(sb)
            else:
                wait_store(sb)

            @pl.loop(0, _LPC, unroll=8)
            def _add_pos(l):
                for h in range(2):
                    for k in range(_F // _LANES):
                        dsl = pl.ds(h * _F + k * _LANES, _LANES)
                        x = rows_v[b, t, 2 * l + h, pl.ds(k * _LANES, _LANES)]
                        store_v[sb, l, dsl] = x + pos_v[t, l, dsl]

            pltpu.async_copy(store_v.at[sb], out_hbm.at[base + j, t],
                             ssem.at[sb])

    start_gather(0, 0)

    @pl.loop(0, _ROWS_PER_W // 2)
    def _row_pair(i):
        j0 = 2 * i
        start_gather(j0 + 1, 1)
        process_row(j0, 0, first=True)

        @pl.when(j0 + 2 < _ROWS_PER_W)
        def _():
            start_gather(j0 + 2, 0)

        process_row(j0 + 1, 1, first=False)

    for sb in range(2):
        wait_store(sb)


_TCB = 32


def _transpose_body(in_ref, out_ref):
    x = in_ref[...].reshape(_TCB, _NLINES, 128)
    i0 = lax.broadcasted_iota(jnp.int32, (_NLINES, _NLINES), 0)
    i1 = lax.broadcasted_iota(jnp.int32, (_NLINES, _NLINES), 1)
    ident = (i0 == i1).astype(jnp.float32)
    xt = lax.dot_general(x, ident, (((1,), (0,)), ((), ())),
                         preferred_element_type=jnp.float32)
    out_ref[:, :, : _S // 2] = xt[:, : _F, :]
    out_ref[:, :, _S // 2:] = xt[:, _F:, :]


def kernel(tokens, node_span_starts, node_span_ends, embed_table, pos_emb):
    del node_span_starts, node_span_ends
    tokens_il = jnp.stack(
        [tokens[:, :_NLINES], tokens[:, _NLINES:]], axis=-1).reshape(_B, _S)
    tokens3 = tokens_il.reshape(_B, _NCHUNK, _CHUNK)
    pos2 = pos_emb.reshape(_S, _F)
    pos_slab = jnp.concatenate(
        [pos2[:_NLINES], pos2[_NLINES:]], axis=1).reshape(_NCHUNK, _LPC, 128)

    mesh = plsc.VectorSubcoreMesh(core_axis_name="c", subcore_axis_name="s")
    lin4 = pl.kernel(
        _embed_body,
        out_type=jax.ShapeDtypeStruct((_B, _NCHUNK, _LPC, 128), jnp.float32),
        mesh=mesh,
        compiler_params=pltpu.CompilerParams(use_tc_tiling_on_sc=False),
        scratch_types=[
            pltpu.VMEM((2, _NCHUNK, _CHUNK), jnp.int32),
            pltpu.VMEM((2, _NCHUNK, _CHUNK, _F), jnp.float32),
            pltpu.VMEM((2, _LPC, 128), jnp.float32),
            pltpu.VMEM((_NCHUNK, _LPC, 128), jnp.float32),
            pltpu.SemaphoreType.DMA((2,)),
            pltpu.SemaphoreType.DMA((2,)),
        ],
    )(tokens3, embed_table, pos_slab)

    out_t = pl.pallas_call(
        _transpose_body,
        grid=(_B // _TCB,),
        in_specs=[pl.BlockSpec((_TCB, _NCHUNK, _LPC, 128),
                               lambda b: (b, 0, 0, 0))],
        out_specs=pl.BlockSpec((_TCB, _F, _S), lambda b: (b, 0, 0)),
        out_shape=jax.ShapeDtypeStruct((_B, _F, _S), jnp.float32),
    )(lin4)
    return out_t.swapaxes(1, 2)


# speedup vs baseline: 3.0729x; 1.0174x over previous
import jax
import jax.numpy as jnp
from jax import lax
from jax.experimental import pallas as pl
from jax.experimental.pallas import tpu as pltpu
from jax.experimental.pallas import tpu_sc as plsc

_B, _S, _F = 1024, 512, 64
_NC, _NS = 2, 16
_NW = _NC * _NS
_ROWS_PER_W = _B // _NW
_CHUNK = 128
_NCHUNK = _S // _CHUNK
_LANES = 16
_NLINES = _S // 2
_LPC = _CHUNK // 2


def _embed_body(tokens_hbm, table_hbm, pos_hbm, out_hbm,
                idx_v, rows_v, store_v, pos_v, gsem, ssem):
    wid = lax.axis_index("s") * _NC + lax.axis_index("c")
    base = wid * _ROWS_PER_W

    pltpu.sync_copy(pos_hbm, pos_v)

    def start_gather(j, b):
        pltpu.sync_copy(tokens_hbm.at[base + j], idx_v.at[b])
        for t in range(_NCHUNK):
            pltpu.async_copy(table_hbm.at[idx_v.at[b, t]], rows_v.at[b, t],
                             gsem.at[b])

    def wait_gather(b):
        for t in range(_NCHUNK):
            pltpu.make_async_copy(table_hbm.at[idx_v.at[b, t]],
                                  rows_v.at[b, t], gsem.at[b]).wait()

    def wait_store(sb):
        pltpu.make_async_copy(store_v.at[sb], out_hbm.at[0, 0],
                              ssem.at[sb]).wait()

    def process_row(j, b, first):
        wait_gather(b)
        for t in range(_NCHUNK):
            sb = t & 1

            if t < 2 and first:
                @pl.when(j > 0)
                def _():
                    wait_store(sb)
            else:
                wait_store(sb)

            @pl.loop(0, _LPC, unroll=4)
            def _add_pos(l):
                for h in range(2):
                    for k in range(_F // _LANES):
                        dsl = pl.ds(h * _F + k * _LANES, _LANES)
                        x = rows_v[b, t, 2 * l + h, pl.ds(k * _LANES, _LANES)]
                        store_v[sb, l, dsl] = x + pos_v[t, l, dsl]

            pltpu.async_copy(store_v.at[sb], out_hbm.at[base + j, t],
                             ssem.at[sb])

    start_gather(0, 0)

    @pl.loop(0, _ROWS_PER_W // 2)
    def _row_pair(i):
        j0 = 2 * i
        start_gather(j0 + 1, 1)
        process_row(j0, 0, first=True)

        @pl.when(j0 + 2 < _ROWS_PER_W)
        def _():
            start_gather(j0 + 2, 0)

        process_row(j0 + 1, 1, first=False)

    for sb in range(2):
        wait_store(sb)


_TCB = 32


def _transpose_body(in_ref, out_ref):
    x = in_ref[...].reshape(_TCB, _NLINES, 128)
    i0 = lax.broadcasted_iota(jnp.int32, (_NLINES, _NLINES), 0)
    i1 = lax.broadcasted_iota(jnp.int32, (_NLINES, _NLINES), 1)
    ident = (i0 == i1).astype(jnp.float32)
    xt = lax.dot_general(x, ident, (((1,), (0,)), ((), ())),
                         preferred_element_type=jnp.float32)
    out_ref[:, :, : _S // 2] = xt[:, : _F, :]
    out_ref[:, :, _S // 2:] = xt[:, _F:, :]


def kernel(tokens, node_span_starts, node_span_ends, embed_table, pos_emb):
    del node_span_starts, node_span_ends
    tokens_il = jnp.stack(
        [tokens[:, :_NLINES], tokens[:, _NLINES:]], axis=-1).reshape(_B, _S)
    tokens3 = tokens_il.reshape(_B, _NCHUNK, _CHUNK)
    pos2 = pos_emb.reshape(_S, _F)
    pos_slab = jnp.concatenate(
        [pos2[:_NLINES], pos2[_NLINES:]], axis=1).reshape(_NCHUNK, _LPC, 128)

    mesh = plsc.VectorSubcoreMesh(core_axis_name="c", subcore_axis_name="s")
    lin4 = pl.kernel(
        _embed_body,
        out_type=jax.ShapeDtypeStruct((_B, _NCHUNK, _LPC, 128), jnp.float32),
        mesh=mesh,
        compiler_params=pltpu.CompilerParams(use_tc_tiling_on_sc=False),
        scratch_types=[
            pltpu.VMEM((2, _NCHUNK, _CHUNK), jnp.int32),
            pltpu.VMEM((2, _NCHUNK, _CHUNK, _F), jnp.float32),
            pltpu.VMEM((2, _LPC, 128), jnp.float32),
            pltpu.VMEM((_NCHUNK, _LPC, 128), jnp.float32),
            pltpu.SemaphoreType.DMA((2,)),
            pltpu.SemaphoreType.DMA((2,)),
        ],
    )(tokens3, embed_table, pos_slab)

    out_t = pl.pallas_call(
        _transpose_body,
        grid=(_B // _TCB,),
        in_specs=[pl.BlockSpec((_TCB, _NCHUNK, _LPC, 128),
                               lambda b: (b, 0, 0, 0))],
        out_specs=pl.BlockSpec((_TCB, _F, _S), lambda b: (b, 0, 0)),
        out_shape=jax.ShapeDtypeStruct((_B, _F, _S), jnp.float32),
    )(lin4)
    return out_t.swapaxes(1, 2)
